# Initial kernel scaffold; baseline (speedup 1.0000x reference)
import jax
import jax.numpy as jnp
from jax.experimental import pallas as pl


def kernel(values, indices, W, b):
    raise NotImplementedError("write your pallas kernel here")


# speedup vs baseline: 2.3194x; 2.3194x over previous
import functools

import jax
import jax.numpy as jnp
from jax import lax
from jax.experimental import pallas as pl
from jax.experimental.pallas import tpu as pltpu
from jax.experimental.pallas import tpu_sc as plsc

N = 16384
NNZ = 262144
D = 64
NC = 2
NS = 16
NW = NC * NS

_MESH = dict(core_axis_name="c", subcore_axis_name="s", num_cores=NC,
             num_subcores=NS)


_SEG_GRP = 1024
_SEG_CHUNK = 128
_ROWS_PER_TILE = NNZ // NS
NH = N // 2


def _seg_pipeline(s, values, idx2d, zeros_hbm, out_hbm, vbuf, wsrc, ibuf,
                  table):
    rows = NH // NS
    zoff = pl.multiple_of(s * rows, rows)
    pltpu.sync_copy(zeros_hbm.at[pl.ds(zoff, rows)],
                    table.at[pl.ds(zoff, rows)])
    plsc.subcore_barrier()

    def chunk(t, _):
        gbase = pl.multiple_of(s * _ROWS_PER_TILE + t * _SEG_GRP, _SEG_GRP)
        pltpu.sync_copy(idx2d.at[pl.ds(pl.multiple_of(gbase // 128, 8),
                                       _SEG_GRP // 128)], ibuf)
        for h in range(_SEG_GRP // _SEG_CHUNK):
            base = pl.multiple_of(gbase + h * _SEG_CHUNK, _SEG_CHUNK)
            pltpu.sync_copy(values.at[pl.ds(base, _SEG_CHUNK)], vbuf)

            def grpf(k, _):
                iv = ibuf[h, pl.ds(k * 16, 16)]
                pv = jnp.bitwise_and(iv, 1)
                hv = lax.shift_right_logical(iv, 1)
                pf = pv.astype(jnp.float32)
                for r in range(16):
                    row = k * 16 + r
                    pfr = lax.gather(
                        pf, jnp.full((16, 1), r, jnp.int32),
                        lax.GatherDimensionNumbers(
                            offset_dims=(), collapsed_slice_dims=(0,),
                            start_index_map=(0,)),
                        (1,),
                        mode=lax.GatherScatterMode.PROMISE_IN_BOUNDS)
                    for cc in range(4):
                        sl = pl.ds(cc * 16, 16)
                        lv = vbuf[row, sl]
                        hi = lv * pfr
                        wsrc[row, sl] = lv - hi
                        wsrc[row, pl.ds(D + cc * 16, 16)] = hi
                pltpu.sync_copy(wsrc.at[pl.ds(k * 16, 16)],
                                table.at[hv], add=True)
                return 0

            lax.fori_loop(0, _SEG_CHUNK // 16, grpf, 0)
        return 0

    lax.fori_loop(0, _ROWS_PER_TILE // _SEG_GRP, chunk, 0)
    plsc.subcore_barrier()
    pltpu.sync_copy(table.at[pl.ds(zoff, rows)],
                    out_hbm.at[pl.ds(zoff, rows)])


def _sc_segsum_body(values, idx3d, zeros_hbm, out_both, vbuf, wsrc, ibuf,
                    table):
    c = lax.axis_index("c")
    s = lax.axis_index("s")
    _seg_pipeline(s, values, idx3d.at[c], zeros_hbm, out_both.at[c], vbuf,
                  wsrc, ibuf, table)


def _sc_segsum(values, idx3d, zeros_hbm):
    return pl.kernel(
        _sc_segsum_body,
        out_type=jax.ShapeDtypeStruct((NC, NH, 2 * D), jnp.float32),
        mesh=plsc.VectorSubcoreMesh(**_MESH),
        scratch_types=[
            pltpu.VMEM((_SEG_CHUNK, D), jnp.float32),
            pltpu.VMEM((_SEG_CHUNK, 2 * D), jnp.float32),
            pltpu.VMEM((_SEG_GRP // 128, 128), jnp.int32),
            pltpu.VMEM_SHARED((NH, 2 * D), jnp.float32),
        ],
    )(values, idx3d, zeros_hbm)


_TC1_BLK = 4096


def _tc1_body(vref, wref, segref, sumref):
    i = pl.program_id(0)
    x = vref[...]
    segref[...] = jnp.dot(x, wref[...], preferred_element_type=jnp.float32)
    colsum = jnp.sum(x, axis=0, keepdims=True)

    @pl.when(i == 0)
    def _():
        sumref[...] = colsum

    @pl.when(i > 0)
    def _():
        sumref[...] += colsum


def _tc1(values, WvT):
    return pl.pallas_call(
        _tc1_body,
        grid=(NNZ // _TC1_BLK,),
        in_specs=[pl.BlockSpec((_TC1_BLK, D), lambda i: (i, 0)),
                  pl.BlockSpec((D, D), lambda i: (0, 0))],
        out_specs=[pl.BlockSpec((_TC1_BLK, D), lambda i: (i, 0)),
                   pl.BlockSpec((1, D), lambda i: (0, 0))],
        out_shape=[jax.ShapeDtypeStruct((NNZ, D), jnp.float32),
                   jax.ShapeDtypeStruct((1, D), jnp.float32)],
    )(values, WvT)


_TC2_BLK = 2048


def _tc2_body(cref, rref, sref, wpr, wpc, wm, bref, about):
    crow = jnp.dot(sref[...] * (1.0 / NNZ), wm[...],
                   preferred_element_type=jnp.float32) + bref[...]
    a = jnp.dot(cref[...], wpr[...],
                preferred_element_type=jnp.float32) + crow
    bb = jnp.dot(rref[...], wpc[...], preferred_element_type=jnp.float32)
    about[...] = jnp.concatenate([a, bb], axis=1)


def _tc2(col_t, row_t, vsum, WprT, WpcT, WmT, b2d):
    small = pl.BlockSpec((D, D), lambda i: (0, 0))
    return pl.pallas_call(
        _tc2_body,
        grid=(N // _TC2_BLK,),
        in_specs=[pl.BlockSpec((_TC2_BLK, D), lambda i: (i, 0)),
                  pl.BlockSpec((_TC2_BLK, D), lambda i: (i, 0)),
                  pl.BlockSpec((1, D), lambda i: (0, 0)),
                  small, small, small,
                  pl.BlockSpec((1, D), lambda i: (0, 0))],
        out_specs=pl.BlockSpec((_TC2_BLK, 2 * D), lambda i: (i, 0)),
        out_shape=jax.ShapeDtypeStruct((N, 2 * D), jnp.float32),
    )(col_t, row_t, vsum, WprT, WpcT, WmT, b2d)


_CMB_GRP = 1024
_CMB_CHUNK = 512
_ROWS_PER_W = NNZ // NW


def _sc_combine_body(seg, AB, col2d, row2d, out, sbuf, abuf, bbuf,
                     icol, irow, sem):
    c = lax.axis_index("c")
    s = lax.axis_index("s")
    w = s * NC + c
    base0 = w * _ROWS_PER_W

    def chunk(t, _):
        gbase = pl.multiple_of(base0 + t * _CMB_GRP, _CMB_GRP)
        ib = pl.multiple_of(gbase // 128, 8)
        pltpu.sync_copy(col2d.at[pl.ds(ib, _CMB_GRP // 128)], icol)
        pltpu.sync_copy(row2d.at[pl.ds(ib, _CMB_GRP // 128)], irow)
        for h in range(_CMB_GRP // _CMB_CHUNK):
            base = pl.multiple_of(gbase + h * _CMB_CHUNK, _CMB_CHUNK)
            pltpu.sync_copy(seg.at[pl.ds(base, _CMB_CHUNK)], sbuf)

            def sub(j, _):
                jj = h * (_CMB_CHUNK // 128) + j
                pltpu.async_copy(AB.at[icol.at[jj]], abuf, sem).wait()
                pltpu.async_copy(AB.at[irow.at[jj]], bbuf, sem).wait()

                def rowf(r, _):
                    for cc in range(4):
                        sl = pl.ds(cc * 16, 16)
                        x = (sbuf[j * 128 + r, sl] + abuf[r, sl]
                             + bbuf[r, pl.ds(D + cc * 16, 16)])
                        sbuf[j * 128 + r, sl] = jnp.maximum(x, 0.01 * x)
                    return 0

                lax.fori_loop(0, 128, rowf, 0)
                return 0

            lax.fori_loop(0, _CMB_CHUNK // 128, sub, 0)
            pltpu.sync_copy(sbuf, out.at[pl.ds(base, _CMB_CHUNK)])
        return 0

    lax.fori_loop(0, _ROWS_PER_W // _CMB_GRP, chunk, 0)


def _sc_combine(seg, AB, col2d, row2d):
    return pl.kernel(
        _sc_combine_body,
        out_type=jax.ShapeDtypeStruct((NNZ, D), jnp.float32),
        mesh=plsc.VectorSubcoreMesh(**_MESH),
        scratch_types=[
            pltpu.VMEM((_CMB_CHUNK, D), jnp.float32),
            pltpu.VMEM((128, 2 * D), jnp.float32),
            pltpu.VMEM((128, 2 * D), jnp.float32),
            pltpu.VMEM((_CMB_GRP // 128, 128), jnp.int32),
            pltpu.VMEM((_CMB_GRP // 128, 128), jnp.int32),
            pltpu.SemaphoreType.DMA,
        ],
    )(seg, AB, col2d, row2d)


def kernel(values, indices, W, b):
    row2d = indices[0].reshape(NNZ // 128, 128)
    col2d = indices[1].reshape(NNZ // 128, 128)
    Wt = W.T
    WvT, WprT, WpcT, WmT = Wt[0:D], Wt[D:2 * D], Wt[2 * D:3 * D], Wt[3 * D:]
    b2d = b.reshape(1, D)

    idx3d = jnp.stack([col2d, row2d])
    zeros_hbm = jnp.zeros((NH, 2 * D), jnp.float32)
    tables = _sc_segsum(values, idx3d, zeros_hbm).reshape(NC, N, D)
    seg, vsum = _tc1(values, WvT)
    AB = _tc2(tables[0], tables[1], vsum, WprT, WpcT, WmT, b2d)
    return _sc_combine(seg, AB, col2d, row2d)


# speedup vs baseline: 2.5592x; 1.1034x over previous
import functools

import jax
import jax.numpy as jnp
from jax import lax
from jax.experimental import pallas as pl
from jax.experimental.pallas import tpu as pltpu
from jax.experimental.pallas import tpu_sc as plsc

N = 16384
NNZ = 262144
D = 64
NC = 2
NS = 16
NW = NC * NS

_MESH = dict(core_axis_name="c", subcore_axis_name="s", num_cores=NC,
             num_subcores=NS)


_SEG_GRP = 1024
_SEG_CHUNK = 128
_ROWS_PER_TILE = NNZ // NS
NH = N // 2


def _seg_pipeline(s, values, idx2d, zeros_hbm, out_hbm, vbuf, wsrc, ibuf,
                  hbuf, table, ssem):
    rows = NH // NS
    zoff = pl.multiple_of(s * rows, rows)
    pltpu.sync_copy(zeros_hbm.at[pl.ds(zoff, rows)],
                    table.at[pl.ds(zoff, rows)])
    plsc.subcore_barrier()

    nchunks = _ROWS_PER_TILE // _SEG_CHUNK

    def chunk(t, _):
        par = jnp.bitwise_and(t, 1)
        h = jnp.bitwise_and(t, 7)
        base = pl.multiple_of(s * _ROWS_PER_TILE + t * _SEG_CHUNK, _SEG_CHUNK)

        @pl.when(h == 0)
        def _():
            pltpu.sync_copy(idx2d.at[pl.ds(pl.multiple_of(base // 128, 8),
                                           8)], ibuf)

        pltpu.sync_copy(values.at[pl.ds(base, _SEG_CHUNK)], vbuf)

        @pl.when(t > 1)
        def _():
            pltpu.make_async_copy(wsrc.at[par],
                                  table.at[hbuf.at[par]], ssem).wait()

        def grpf(k, _):
            iv = ibuf[h, pl.ds(k * 16, 16)]
            hv = lax.shift_right_logical(iv, 1)
            hbuf[par, pl.ds(k * 16, 16)] = hv
            pf = jnp.bitwise_and(iv, 1).astype(jnp.float32)
            for r in range(16):
                row = k * 16 + r
                pfr = lax.gather(
                    pf, jnp.full((16, 1), r, jnp.int32),
                    lax.GatherDimensionNumbers(
                        offset_dims=(), collapsed_slice_dims=(0,),
                        start_index_map=(0,)),
                    (1,),
                    mode=lax.GatherScatterMode.PROMISE_IN_BOUNDS)
                for cc in range(4):
                    sl = pl.ds(cc * 16, 16)
                    lv = vbuf[row, sl]
                    hi = lv * pfr
                    wsrc[par, row, sl] = lv - hi
                    wsrc[par, row, pl.ds(D + cc * 16, 16)] = hi
            return 0

        lax.fori_loop(0, _SEG_CHUNK // 16, grpf, 0)
        pltpu.async_copy(wsrc.at[par], table.at[hbuf.at[par]], ssem, add=True)
        return 0

    lax.fori_loop(0, nchunks, chunk, 0)
    pltpu.make_async_copy(wsrc.at[0], table.at[hbuf.at[0]], ssem).wait()
    pltpu.make_async_copy(wsrc.at[1], table.at[hbuf.at[1]], ssem).wait()
    plsc.subcore_barrier()
    pltpu.sync_copy(table.at[pl.ds(zoff, rows)],
                    out_hbm.at[pl.ds(zoff, rows)])


def _sc_segsum_body(values, idx3d, zeros_hbm, out_both, vbuf, wsrc, ibuf,
                    hbuf, table, ssem):
    c = lax.axis_index("c")
    s = lax.axis_index("s")
    _seg_pipeline(s, values, idx3d.at[c], zeros_hbm, out_both.at[c], vbuf,
                  wsrc, ibuf, hbuf, table, ssem)


def _sc_segsum(values, idx3d, zeros_hbm):
    return pl.kernel(
        _sc_segsum_body,
        out_type=jax.ShapeDtypeStruct((NC, NH, 2 * D), jnp.float32),
        mesh=plsc.VectorSubcoreMesh(**_MESH),
        scratch_types=[
            pltpu.VMEM((_SEG_CHUNK, D), jnp.float32),
            pltpu.VMEM((2, _SEG_CHUNK, 2 * D), jnp.float32),
            pltpu.VMEM((_SEG_GRP // 128, 128), jnp.int32),
            pltpu.VMEM((2, 128), jnp.int32),
            pltpu.VMEM_SHARED((NH, 2 * D), jnp.float32),
            pltpu.SemaphoreType.DMA,
        ],
    )(values, idx3d, zeros_hbm)


_TC1_BLK = 4096


def _tc1_body(vref, wref, segref, sumref):
    i = pl.program_id(0)
    x = vref[...]
    segref[...] = jnp.dot(x, wref[...], preferred_element_type=jnp.float32)
    colsum = jnp.sum(x, axis=0, keepdims=True)

    @pl.when(i == 0)
    def _():
        sumref[...] = colsum

    @pl.when(i > 0)
    def _():
        sumref[...] += colsum


def _tc1(values, WvT):
    return pl.pallas_call(
        _tc1_body,
        grid=(NNZ // _TC1_BLK,),
        in_specs=[pl.BlockSpec((_TC1_BLK, D), lambda i: (i, 0)),
                  pl.BlockSpec((D, D), lambda i: (0, 0))],
        out_specs=[pl.BlockSpec((_TC1_BLK, D), lambda i: (i, 0)),
                   pl.BlockSpec((1, D), lambda i: (0, 0))],
        out_shape=[jax.ShapeDtypeStruct((NNZ, D), jnp.float32),
                   jax.ShapeDtypeStruct((1, D), jnp.float32)],
    )(values, WvT)


_TC2_BLK = 2048


def _tc2_body(cref, rref, sref, wpr, wpc, wm, bref, about):
    crow = jnp.dot(sref[...] * (1.0 / NNZ), wm[...],
                   preferred_element_type=jnp.float32) + bref[...]
    a = jnp.dot(cref[...], wpr[...],
                preferred_element_type=jnp.float32) + crow
    bb = jnp.dot(rref[...], wpc[...], preferred_element_type=jnp.float32)
    about[...] = jnp.concatenate([a, bb], axis=1)


def _tc2(col_t, row_t, vsum, WprT, WpcT, WmT, b2d):
    small = pl.BlockSpec((D, D), lambda i: (0, 0))
    return pl.pallas_call(
        _tc2_body,
        grid=(N // _TC2_BLK,),
        in_specs=[pl.BlockSpec((_TC2_BLK, D), lambda i: (i, 0)),
                  pl.BlockSpec((_TC2_BLK, D), lambda i: (i, 0)),
                  pl.BlockSpec((1, D), lambda i: (0, 0)),
                  small, small, small,
                  pl.BlockSpec((1, D), lambda i: (0, 0))],
        out_specs=pl.BlockSpec((_TC2_BLK, 2 * D), lambda i: (i, 0)),
        out_shape=jax.ShapeDtypeStruct((N, 2 * D), jnp.float32),
    )(col_t, row_t, vsum, WprT, WpcT, WmT, b2d)


_CMB_GRP = 1024
_CMB_CHUNK = 256
_ROWS_PER_W = NNZ // NW


def _sc_combine_body(seg, AB, col2d, row2d, out, sbuf, abuf, bbuf,
                     icol, irow, sema, semb, sems):
    c = lax.axis_index("c")
    s = lax.axis_index("s")
    w = s * NC + c
    base0 = w * _ROWS_PER_W

    def unit(u, _):
        base = pl.multiple_of(base0 + u * _CMB_CHUNK, _CMB_CHUNK)

        @pl.when(jnp.bitwise_and(u, 3) == 0)
        def _():
            ib = pl.multiple_of(base // 128, 8)
            pltpu.sync_copy(col2d.at[pl.ds(ib, _CMB_GRP // 128)], icol)
            pltpu.sync_copy(row2d.at[pl.ds(ib, _CMB_GRP // 128)], irow)

        dseg = pltpu.async_copy(seg.at[pl.ds(base, _CMB_CHUNK)], sbuf, sems)
        hrow = jnp.bitwise_and(u, 3) * 2
        d = []
        for j in range(2):
            d.append((pltpu.async_copy(AB.at[icol.at[hrow + j]],
                                       abuf.at[j], sema),
                      pltpu.async_copy(AB.at[irow.at[hrow + j]],
                                       bbuf.at[j], semb)))
        dseg.wait()
        for j in range(2):
            d[j][0].wait()
            d[j][1].wait()

            def rowf(r, _):
                for cc in range(4):
                    sl = pl.ds(cc * 16, 16)
                    x = (sbuf[j * 128 + r, sl] + abuf[j, r, sl]
                         + bbuf[j, r, pl.ds(D + cc * 16, 16)])
                    sbuf[j * 128 + r, sl] = jnp.maximum(x, 0.01 * x)
                return 0

            lax.fori_loop(0, 128, rowf, 0)
        pltpu.sync_copy(sbuf, out.at[pl.ds(base, _CMB_CHUNK)])
        return 0

    lax.fori_loop(0, _ROWS_PER_W // _CMB_CHUNK, unit, 0)


def _sc_combine(seg, AB, col2d, row2d):
    return pl.kernel(
        _sc_combine_body,
        out_type=jax.ShapeDtypeStruct((NNZ, D), jnp.float32),
        mesh=plsc.VectorSubcoreMesh(**_MESH),
        scratch_types=[
            pltpu.VMEM((_CMB_CHUNK, D), jnp.float32),
            pltpu.VMEM((2, 128, 2 * D), jnp.float32),
            pltpu.VMEM((2, 128, 2 * D), jnp.float32),
            pltpu.VMEM((_CMB_GRP // 128, 128), jnp.int32),
            pltpu.VMEM((_CMB_GRP // 128, 128), jnp.int32),
            pltpu.SemaphoreType.DMA,
            pltpu.SemaphoreType.DMA,
            pltpu.SemaphoreType.DMA,
        ],
    )(seg, AB, col2d, row2d)


def kernel(values, indices, W, b):
    row2d = indices[0].reshape(NNZ // 128, 128)
    col2d = indices[1].reshape(NNZ // 128, 128)
    Wt = W.T
    WvT, WprT, WpcT, WmT = Wt[0:D], Wt[D:2 * D], Wt[2 * D:3 * D], Wt[3 * D:]
    b2d = b.reshape(1, D)

    idx3d = jnp.stack([col2d, row2d])
    zeros_hbm = jnp.zeros((NH, 2 * D), jnp.float32)
    tables = _sc_segsum(values, idx3d, zeros_hbm).reshape(NC, N, D)
    seg, vsum = _tc1(values, WvT)
    AB = _tc2(tables[0], tables[1], vsum, WprT, WpcT, WmT, b2d)
    return _sc_combine(seg, AB, col2d, row2d)
